# Initial kernel scaffold; baseline (speedup 1.0000x reference)
import jax
import jax.numpy as jnp
from jax.experimental import pallas as pl


def kernel(x, batch, params):
    raise NotImplementedError("write your pallas kernel here")


# speedup vs baseline: 18.5348x; 18.5348x over previous
import functools

import jax
import jax.numpy as jnp
from jax.experimental import pallas as pl
from jax.experimental.pallas import tpu as pltpu

N = 50000
G = 512
IN_DIM = 128
HID = 32
FEAT = 64
NAGG = 8
EPS = 1e-5

C = 1024
SB = 128
NB = (N + C - 1) // C
NPAD = NB * C
NEG = -1e30


def _norm(h, mv_ref, gw_ref, gb_ref):
    mv = mv_ref[...]
    m = mv[0:1, 0:1]
    v = mv[0:1, 1:2]
    return (h - m) / jnp.sqrt(v + EPS) * gw_ref[...] + gb_ref[...]


def _input_mm_kernel(x_ref, w_ref, b_ref, o_ref):
    h = jnp.dot(x_ref[...], w_ref[...], preferred_element_type=jnp.float32)
    o_ref[...] = h + b_ref[...]


def _input_mm(xpad, p):
    return pl.pallas_call(
        _input_mm_kernel,
        grid=(NB,),
        in_specs=[
            pl.BlockSpec((C, IN_DIM), lambda i: (i, 0)),
            pl.BlockSpec((IN_DIM, HID), lambda i: (0, 0)),
            pl.BlockSpec((1, HID), lambda i: (0, 0)),
        ],
        out_specs=pl.BlockSpec((C, HID), lambda i: (i, 0)),
        out_shape=jax.ShapeDtypeStruct((NPAD, HID), jnp.float32),
    )(xpad, p["lin"]["W"], p["lin"]["b"].reshape(1, HID))


def _rowln_kernel(h_ref, m_ref, v_ref, lw_ref, lb_ref, o_ref):
    o_ref[...] = jnp.tanh(
        (h_ref[...] - m_ref[...]) / jnp.sqrt(v_ref[...] + EPS)
        * lw_ref[...] + lb_ref[...])


def _rowln(hlin, mpad, vpad, p):
    return pl.pallas_call(
        _rowln_kernel,
        grid=(NB,),
        in_specs=[
            pl.BlockSpec((C, HID), lambda i: (i, 0)),
            pl.BlockSpec((C, 1), lambda i: (i, 0)),
            pl.BlockSpec((C, 1), lambda i: (i, 0)),
            pl.BlockSpec((1, HID), lambda i: (0, 0)),
            pl.BlockSpec((1, HID), lambda i: (0, 0)),
        ],
        out_specs=pl.BlockSpec((C, HID), lambda i: (i, 0)),
        out_shape=jax.ShapeDtypeStruct((NPAD, HID), jnp.float32),
    )(hlin, mpad, vpad, p["ln_w"].reshape(1, HID), p["ln_b"].reshape(1, HID))


def _passA_kernel(meta_ref, h_ref, mv_ref, gw_ref, gb_ref,
                  w1_ref, b1_ref, ws_ref, bs_ref,
                  S_ref, M_ref, xp_scr, at_scr):
    i = pl.program_id(0)

    @pl.when(i == 0)
    def _():
        S_ref[...] = jnp.zeros_like(S_ref)
        M_ref[...] = jnp.full_like(M_ref, NEG)

    hn = _norm(h_ref[...], mv_ref, gw_ref, gb_ref)
    xp = jnp.dot(hn, w1_ref[...], preferred_element_type=jnp.float32)
    xp = xp + b1_ref[...]
    sc = jnp.dot(xp, ws_ref[...], preferred_element_type=jnp.float32)
    at = jnp.exp(-jnp.abs(sc + bs_ref[...]))
    xp_scr[...] = xp
    at_scr[...] = at

    base = i * C
    g_lo = meta_ref[G + 1 + i]
    g_hi = meta_ref[G + 1 + NB + i]
    iota = jax.lax.broadcasted_iota(jnp.int32, (SB, 1), 0)

    def gbody(g, carry):
        s = jnp.maximum(meta_ref[g] - base, 0)
        e = jnp.minimum(meta_ref[g + 1] - base, C)
        k0 = s // SB
        k1 = (e + SB - 1) // SB

        def kbody(k, kc):
            Sa, Ma = kc
            r0 = k * SB
            sub_at = at_scr[pl.ds(r0, SB), :]
            sub_xp = xp_scr[pl.ds(r0, SB), :]
            rows = iota + r0
            mask = (rows >= s) & (rows < e)
            s_rows = []
            pm_rows = []
            for a in range(NAGG):
                prod = sub_at[:, a:a + 1] * sub_xp
                s_rows.append(jnp.sum(jnp.where(mask, prod, 0.0), axis=0,
                                      keepdims=True))
                pm_rows.append(jnp.max(jnp.where(mask, prod, NEG), axis=0,
                                       keepdims=True))
            Sa = Sa + jnp.concatenate(s_rows, axis=0)
            Ma = jnp.maximum(Ma, jnp.concatenate(pm_rows, axis=0))
            return Sa, Ma

        Sa, Ma = jax.lax.fori_loop(
            k0, k1, kbody,
            (jnp.zeros((NAGG, FEAT), jnp.float32),
             jnp.full((NAGG, FEAT), NEG, jnp.float32)))
        r = pl.ds(g * NAGG, NAGG)
        S_ref[r, :] = S_ref[r, :] + Sa
        M_ref[r, :] = jnp.maximum(M_ref[r, :], Ma)
        return carry

    jax.lax.fori_loop(g_lo, g_hi + 1, gbody, 0)


def _passA(meta, h, mv, gw, gb, w1, b1, ws, bs):
    grid_spec = pltpu.PrefetchScalarGridSpec(
        num_scalar_prefetch=1,
        grid=(NB,),
        in_specs=[
            pl.BlockSpec((C, HID), lambda i, m: (i, 0)),
            pl.BlockSpec((1, 128), lambda i, m: (0, 0)),
            pl.BlockSpec((1, HID), lambda i, m: (0, 0)),
            pl.BlockSpec((1, HID), lambda i, m: (0, 0)),
            pl.BlockSpec((HID, FEAT), lambda i, m: (0, 0)),
            pl.BlockSpec((1, FEAT), lambda i, m: (0, 0)),
            pl.BlockSpec((FEAT, NAGG), lambda i, m: (0, 0)),
            pl.BlockSpec((1, NAGG), lambda i, m: (0, 0)),
        ],
        out_specs=[
            pl.BlockSpec((G * NAGG, FEAT), lambda i, m: (0, 0)),
            pl.BlockSpec((G * NAGG, FEAT), lambda i, m: (0, 0)),
        ],
        scratch_shapes=[
            pltpu.VMEM((C, FEAT), jnp.float32),
            pltpu.VMEM((C, NAGG), jnp.float32),
        ],
    )
    return pl.pallas_call(
        _passA_kernel,
        grid_spec=grid_spec,
        out_shape=[
            jax.ShapeDtypeStruct((G * NAGG, FEAT), jnp.float32),
            jax.ShapeDtypeStruct((G * NAGG, FEAT), jnp.float32),
        ],
    )(meta, h, mv, gw, gb, w1, b1, ws, bs)


def _passB_kernel(meta_ref, h_ref, mv_ref, gw_ref, gb_ref,
                  S2_ref, M2_ref, cnt_ref, wm_ref, wx_ref,
                  w1_ref, b1_ref, wh_ref, wxp_ref, b2_ref,
                  ho_ref, hs_ref, st_ref, garr_scr, *, want_hseg):
    i = pl.program_id(0)

    @pl.when(i == 0)
    def _():
        mean = S2_ref[...] / cnt_ref[...]
        mx = M2_ref[...]
        mx = jnp.where(mx > 0.5 * NEG, mx, 0.0)
        garr_scr[...] = (
            jnp.dot(mean, wm_ref[...], preferred_element_type=jnp.float32)
            + jnp.dot(mx, wx_ref[...], preferred_element_type=jnp.float32)
            + b2_ref[...])
        hs_ref[...] = jnp.zeros_like(hs_ref)
        st_ref[...] = jnp.zeros_like(st_ref)

    hn = _norm(h_ref[...], mv_ref, gw_ref, gb_ref)
    xp = jnp.dot(hn, w1_ref[...], preferred_element_type=jnp.float32)
    xp = xp + b1_ref[...]
    acc = (jnp.dot(hn, wh_ref[...], preferred_element_type=jnp.float32)
           + jnp.dot(xp, wxp_ref[...], preferred_element_type=jnp.float32))

    base = i * C
    g_lo = meta_ref[G + 1 + i]
    g_hi = meta_ref[G + 1 + NB + i]
    iota = jax.lax.broadcasted_iota(jnp.int32, (C, 1), 0)
    iota8 = jax.lax.broadcasted_iota(jnp.int32, (8, 1), 0)

    def gbody(g, acc):
        s = jnp.maximum(meta_ref[g] - base, 0)
        e = jnp.minimum(meta_ref[g + 1] - base, C)
        mask = (iota >= s) & (iota < e)
        blk = garr_scr[pl.ds((g // 8) * 8, 8), :]
        grow = jnp.sum(jnp.where(iota8 == g % 8, blk, 0.0), axis=0,
                       keepdims=True)
        return acc + jnp.where(mask, grow, 0.0)

    acc = jax.lax.fori_loop(g_lo, g_hi + 1, gbody, acc)
    hb = jnp.tanh(acc)
    ho_ref[...] = hb

    gi = iota + base
    hv = jnp.where(gi < N, hb, 0.0)
    srow = jnp.sum(hv, axis=0, keepdims=True)
    sqrow = jnp.sum(hv * hv, axis=0, keepdims=True)
    st_ref[0:1, 0:HID] = st_ref[0:1, 0:HID] + srow
    st_ref[1:2, 0:HID] = st_ref[1:2, 0:HID] + sqrow

    if want_hseg:
        def g2body(g, carry):
            s = jnp.maximum(meta_ref[g] - base, 0)
            e = jnp.minimum(meta_ref[g + 1] - base, C)
            mask = (iota >= s) & (iota < e)
            part = jnp.sum(jnp.where(mask, hb, 0.0), axis=0, keepdims=True)
            a0 = (g // 8) * 8
            sel = iota8 == g % 8
            hs_ref[pl.ds(a0, 8), :] = (hs_ref[pl.ds(a0, 8), :]
                                       + jnp.where(sel, part, 0.0))
            return carry

        jax.lax.fori_loop(g_lo, g_hi + 1, g2body, 0)


def _passB(meta, h, mv, gw, gb, S2, M2, cnt, wm, wx, w1, b1, wh, wxp, b2,
           want_hseg):
    grid_spec = pltpu.PrefetchScalarGridSpec(
        num_scalar_prefetch=1,
        grid=(NB,),
        in_specs=[
            pl.BlockSpec((C, HID), lambda i, m: (i, 0)),
            pl.BlockSpec((1, 128), lambda i, m: (0, 0)),
            pl.BlockSpec((1, HID), lambda i, m: (0, 0)),
            pl.BlockSpec((1, HID), lambda i, m: (0, 0)),
            pl.BlockSpec((G, NAGG * FEAT), lambda i, m: (0, 0)),
            pl.BlockSpec((G, NAGG * FEAT), lambda i, m: (0, 0)),
            pl.BlockSpec((G, 1), lambda i, m: (0, 0)),
            pl.BlockSpec((NAGG * FEAT, HID), lambda i, m: (0, 0)),
            pl.BlockSpec((NAGG * FEAT, HID), lambda i, m: (0, 0)),
            pl.BlockSpec((HID, FEAT), lambda i, m: (0, 0)),
            pl.BlockSpec((1, FEAT), lambda i, m: (0, 0)),
            pl.BlockSpec((HID, HID), lambda i, m: (0, 0)),
            pl.BlockSpec((FEAT, HID), lambda i, m: (0, 0)),
            pl.BlockSpec((1, HID), lambda i, m: (0, 0)),
        ],
        out_specs=[
            pl.BlockSpec((C, HID), lambda i, m: (i, 0)),
            pl.BlockSpec((G, HID), lambda i, m: (0, 0)),
            pl.BlockSpec((8, 128), lambda i, m: (0, 0)),
        ],
        scratch_shapes=[pltpu.VMEM((G, HID), jnp.float32)],
    )
    return pl.pallas_call(
        functools.partial(_passB_kernel, want_hseg=want_hseg),
        grid_spec=grid_spec,
        out_shape=[
            jax.ShapeDtypeStruct((NPAD, HID), jnp.float32),
            jax.ShapeDtypeStruct((G, HID), jnp.float32),
            jax.ShapeDtypeStruct((8, 128), jnp.float32),
        ],
    )(meta, h, mv, gw, gb, S2, M2, cnt, wm, wx, w1, b1, wh, wxp, b2)


def _head_kernel(hs_ref, cnt_ref, mv_ref, gw_ref, gb_ref,
                 w0, b0, lw0, lb0, w1, b1, lw1, lb1, w2, b2, lw2, lb2,
                 wf, bf, o_ref):
    s = _norm(hs_ref[...] / cnt_ref[...], mv_ref, gw_ref, gb_ref)
    for w, b, lw, lb in ((w0, b0, lw0, lb0), (w1, b1, lw1, lb1),
                         (w2, b2, lw2, lb2)):
        t = jnp.dot(s, w[...], preferred_element_type=jnp.float32) + b[...]
        m = jnp.mean(t, axis=1, keepdims=True)
        v = jnp.mean((t - m) ** 2, axis=1, keepdims=True)
        s = jnp.tanh((t - m) / jnp.sqrt(v + EPS) * lw[...] + lb[...])
    out = jnp.sum(s * wf[...], axis=1, keepdims=True)
    o_ref[...] = out + bf[...]


def _head(hseg, cnt, mv, gw, gb, pred):
    args = [hseg, cnt, mv, gw, gb]
    in_specs = [
        pl.BlockSpec((G, HID), lambda: (0, 0)),
        pl.BlockSpec((G, 1), lambda: (0, 0)),
        pl.BlockSpec((1, 128), lambda: (0, 0)),
        pl.BlockSpec((1, HID), lambda: (0, 0)),
        pl.BlockSpec((1, HID), lambda: (0, 0)),
    ]
    for p in pred["hid"]:
        args += [p["lin"]["W"], p["lin"]["b"].reshape(1, HID),
                 p["ln_w"].reshape(1, HID), p["ln_b"].reshape(1, HID)]
        in_specs += [pl.BlockSpec((HID, HID), lambda: (0, 0)),
                     pl.BlockSpec((1, HID), lambda: (0, 0)),
                     pl.BlockSpec((1, HID), lambda: (0, 0)),
                     pl.BlockSpec((1, HID), lambda: (0, 0))]
    args += [pred["fin"]["W"].reshape(1, HID),
             jnp.broadcast_to(pred["fin"]["b"].reshape(1, 1), (1, 128))]
    in_specs += [pl.BlockSpec((1, HID), lambda: (0, 0)),
                 pl.BlockSpec((1, 128), lambda: (0, 0))]
    return pl.pallas_call(
        _head_kernel,
        in_specs=in_specs,
        out_specs=pl.BlockSpec((G, 128), lambda: (0, 0)),
        out_shape=jax.ShapeDtypeStruct((G, 128), jnp.float32),
    )(*args)


def _pack_mv(m, v):
    return jnp.zeros((1, 128), jnp.float32).at[0, 0].set(m).at[0, 1].set(v)


def kernel(x, batch, params):
    batch = batch.astype(jnp.int32)
    offsets = jnp.searchsorted(
        batch, jnp.arange(G + 1, dtype=jnp.int32)).astype(jnp.int32)
    cnt = jnp.maximum(
        (offsets[1:] - offsets[:-1]).astype(jnp.float32), 1.0).reshape(G, 1)
    starts = jnp.arange(NB, dtype=jnp.int32) * C
    ends = jnp.minimum(starts + C, N) - 1
    meta = jnp.concatenate([offsets, batch[starts], batch[ends]])
    xpad = jnp.pad(x, ((0, NPAD - N), (0, 0)))

    hlin = _input_mm(xpad, params["inp"])
    hr = hlin[:N]
    m0 = hr.mean(axis=-1, keepdims=True)
    v0 = ((hr - m0) ** 2).mean(axis=-1, keepdims=True)
    mpad = jnp.pad(m0, ((0, NPAD - N), (0, 0)))
    vpad = jnp.pad(v0, ((0, NPAD - N), (0, 0)))
    h = _rowln(hlin, mpad, vpad, params["inp"])

    ones = jnp.ones((1, HID), jnp.float32)
    zeros = jnp.zeros((1, HID), jnp.float32)
    mv = _pack_mv(0.0, 1.0 - EPS)
    gw, gb = ones, zeros
    hseg = None
    for it in range(9):
        p = params["bip"][it % 3]
        W3 = p["t_out"]["W"]
        Wh = W3[:HID]
        Wxp = W3[HID:HID + FEAT]
        Wagg = W3[HID + FEAT:].reshape(NAGG, 2 * FEAT, HID)
        Wm = Wagg[:, :FEAT, :].reshape(NAGG * FEAT, HID)
        Wx = Wagg[:, FEAT:, :].reshape(NAGG * FEAT, HID)
        W1 = p["t_in"]["W"]
        b1 = p["t_in"]["b"].reshape(1, FEAT)
        b2 = p["t_out"]["b"].reshape(1, HID)
        ws = p["score"]["W"]
        bs = p["score"]["b"].reshape(1, NAGG)

        S, M = _passA(meta, h, mv, gw, gb, W1, b1, ws, bs)
        S2 = S.reshape(G, NAGG * FEAT)
        M2 = M.reshape(G, NAGG * FEAT)
        h, hseg, st = _passB(meta, h, mv, gw, gb, S2, M2, cnt, Wm, Wx,
                             W1, b1, Wh, Wxp, b2, it == 8)

        hcur = h[:N]
        m = hcur.mean()
        v = ((hcur - m) ** 2).mean()
        mv = _pack_mv(m, v)
        gw = p["gln_w"].reshape(1, HID)
        gb = p["gln_b"].reshape(1, HID)

    out = _head(hseg, cnt, mv, gw, gb, params["pred"])
    return out[:, 0]
